# baseline (device time: 529554 ns/iter reference)
import jax
import jax.numpy as jnp
from jax import lax
from jax.experimental import pallas as pl
from jax.experimental.pallas import tpu as pltpu

N_DEV = 4
STRIP = 256


def kernel(A, B):
    m_per, k = A.shape
    _, n = B.shape
    m_half = m_per // 2
    n_strips = m_half // STRIP
    n_pairs = n_strips // 2

    A16 = A.astype(jnp.bfloat16)
    B16 = B.astype(jnp.bfloat16)

    def body(a_ref, b_ref, out_ref,
             cw_ref, ccw_ref, c0_ref, c1_ref,
             in_sems, out_sems,
             send_cw, recv_cw, send_ccw, recv_ccw,
             psend_cw, precv_cw, psend_ccw, precv_ccw):
        my = lax.axis_index("i")
        left = (my - 1) % N_DEV
        right = (my + 1) % N_DEV

        cp_top = pltpu.make_async_copy(
            a_ref.at[pl.ds(0, m_half), :], cw_ref.at[0], in_sems.at[0]
        )
        cp_bot = pltpu.make_async_copy(
            a_ref.at[pl.ds(m_half, m_half), :], ccw_ref.at[0], in_sems.at[1]
        )
        cp_top.start()
        cp_bot.start()

        barrier_sem = pltpu.get_barrier_semaphore()
        for nbr in (left, right):
            pl.semaphore_signal(
                barrier_sem, inc=1,
                device_id=(nbr,), device_id_type=pl.DeviceIdType.MESH,
            )
        pl.semaphore_wait(barrier_sem, 2)
        cp_top.wait()
        cp_bot.wait()

        def out_desc(c_ref, sem, row):
            return pltpu.make_async_copy(
                c_ref, out_ref.at[pl.ds(row, STRIP), :], sem
            )

        def compute_span(comm_ref, slot, src_row0, out_row0, strips):
            def pair_step(p, _):
                s0 = 2 * p

                @pl.when(p >= 1)
                def _():
                    out_desc(c0_ref, out_sems.at[0], out_row0).wait()

                c0_ref[...] = jnp.dot(
                    comm_ref[slot, pl.ds(src_row0 + s0 * STRIP, STRIP), :],
                    b_ref[...], preferred_element_type=jnp.float32,
                ).astype(jnp.bfloat16)
                out_desc(c0_ref, out_sems.at[0], out_row0 + s0 * STRIP).start()

                @pl.when(p >= 1)
                def _():
                    out_desc(c1_ref, out_sems.at[1], out_row0).wait()

                c1_ref[...] = jnp.dot(
                    comm_ref[slot, pl.ds(src_row0 + (s0 + 1) * STRIP, STRIP), :],
                    b_ref[...], preferred_element_type=jnp.float32,
                ).astype(jnp.bfloat16)
                out_desc(c1_ref, out_sems.at[1], out_row0 + (s0 + 1) * STRIP).start()
                return _

            lax.fori_loop(0, strips // 2, pair_step, None)
            out_desc(c0_ref, out_sems.at[0], out_row0).wait()
            out_desc(c1_ref, out_sems.at[1], out_row0).wait()

        def compute_half(comm_ref, slot, row_base):
            compute_span(comm_ref, slot, 0, row_base, n_strips)

        m_piece = m_half // 2
        piece_strips = m_piece // STRIP

        def hop_step(h, _):
            slot = h % 2
            rdma_cw = pltpu.make_async_remote_copy(
                src_ref=cw_ref.at[slot],
                dst_ref=cw_ref.at[1 - slot],
                send_sem=send_cw.at[slot],
                recv_sem=recv_cw.at[1 - slot],
                device_id=(right,),
                device_id_type=pl.DeviceIdType.MESH,
            )
            rdma_ccw = pltpu.make_async_remote_copy(
                src_ref=ccw_ref.at[slot],
                dst_ref=ccw_ref.at[1 - slot],
                send_sem=send_ccw.at[slot],
                recv_sem=recv_ccw.at[1 - slot],
                device_id=(left,),
                device_id_type=pl.DeviceIdType.MESH,
            )
            rdma_cw.start()
            rdma_ccw.start()

            origin_cw = (my - h) % N_DEV
            origin_ccw = (my + h) % N_DEV
            compute_half(cw_ref, slot, origin_cw * m_per)
            compute_half(ccw_ref, slot, origin_ccw * m_per + m_half)

            rdma_cw.wait()
            rdma_ccw.wait()
            return _

        lax.fori_loop(0, 2, hop_step, None)

        def piece_rdma(comm_ref, q, ssem, rsem, dev):
            rows = pl.ds(q * m_piece, m_piece)
            return pltpu.make_async_remote_copy(
                src_ref=comm_ref.at[0, rows, :],
                dst_ref=comm_ref.at[1, rows, :],
                send_sem=ssem.at[q],
                recv_sem=rsem.at[q],
                device_id=(dev,),
                device_id_type=pl.DeviceIdType.MESH,
            )

        h2 = [
            piece_rdma(cw_ref, 0, psend_cw, precv_cw, right),
            piece_rdma(ccw_ref, 0, psend_ccw, precv_ccw, left),
            piece_rdma(cw_ref, 1, psend_cw, precv_cw, right),
            piece_rdma(ccw_ref, 1, psend_ccw, precv_ccw, left),
        ]
        for r in h2:
            r.start()

        origin_cw = (my - 2) % N_DEV
        origin_ccw = (my + 2) % N_DEV
        compute_half(cw_ref, 0, origin_cw * m_per)
        compute_half(ccw_ref, 0, origin_ccw * m_per + m_half)
        for r in h2:
            r.wait_send()

        origin_cw = (my - 3) % N_DEV
        origin_ccw = (my + 3) % N_DEV

        def piece_step(piece, _):
            piece_rdma(cw_ref, piece, psend_cw, precv_cw, left).wait_recv()
            compute_span(
                cw_ref, 1, piece * m_piece,
                origin_cw * m_per + piece * m_piece, piece_strips,
            )
            piece_rdma(ccw_ref, piece, psend_ccw, precv_ccw, right).wait_recv()
            compute_span(
                ccw_ref, 1, piece * m_piece,
                origin_ccw * m_per + m_half + piece * m_piece, piece_strips,
            )
            return _

        lax.fori_loop(0, 2, piece_step, None)

    return pl.pallas_call(
        body,
        out_shape=jax.ShapeDtypeStruct((N_DEV * m_per, n), jnp.bfloat16),
        in_specs=[
            pl.BlockSpec(memory_space=pl.ANY),
            pl.BlockSpec(memory_space=pltpu.VMEM),
        ],
        out_specs=pl.BlockSpec(memory_space=pltpu.MemorySpace.HBM),
        scratch_shapes=[
            pltpu.VMEM((2, m_per // 2, k), jnp.bfloat16),
            pltpu.VMEM((2, m_per // 2, k), jnp.bfloat16),
            pltpu.VMEM((STRIP, n), jnp.bfloat16),
            pltpu.VMEM((STRIP, n), jnp.bfloat16),
            pltpu.SemaphoreType.DMA((2,)),
            pltpu.SemaphoreType.DMA((2,)),
            pltpu.SemaphoreType.DMA((2,)),
            pltpu.SemaphoreType.DMA((2,)),
            pltpu.SemaphoreType.DMA((2,)),
            pltpu.SemaphoreType.DMA((2,)),
            pltpu.SemaphoreType.DMA((2,)),
            pltpu.SemaphoreType.DMA((2,)),
            pltpu.SemaphoreType.DMA((2,)),
            pltpu.SemaphoreType.DMA((2,)),
        ],
        compiler_params=pltpu.CompilerParams(
            collective_id=0, vmem_limit_bytes=64 * 1024 * 1024
        ),
    )(A16, B16).astype(jnp.float32)
